# baseline (speedup 1.0000x reference)
import functools

import jax
import jax.numpy as jnp
from jax import lax
from jax.experimental import pallas as pl
from jax.experimental.pallas import tpu as pltpu
from jax.experimental.pallas import tpu_sc as plsc

NUM_EMBEDDINGS = 1000000
EMBEDDING_DIM = 64
BATCH = 16384
HIST = 50

NC, NS = 2, 16
NW = NC * NS
D = EMBEDDING_DIM
BB = 128
BPW = BATCH // NW // BB
NBLK = BPW * HIST
HPAD = 56

_mesh = plsc.VectorSubcoreMesh(core_axis_name="c", subcore_axis_name="s",
                               num_cores=NC, num_subcores=NS)


@functools.partial(
    pl.kernel,
    out_type=jax.ShapeDtypeStruct((HIST * D // 8, BATCH // BB, 8, BB),
                                  jnp.float32),
    mesh=_mesh,
    compiler_params=pltpu.CompilerParams(needs_layout_passes=False),
    scratch_types=[
        pltpu.VMEM((8, BPW * BB), jnp.int32),
        pltpu.VMEM((2, BB), jnp.int32),
        pltpu.VMEM((2, 8, 16), jnp.int32),
        pltpu.VMEM((2, BB, 2 * D), jnp.float32),
        pltpu.VMEM((2, 8, 8, BB), jnp.float32),
        pltpu.SemaphoreType.DMA,
        pltpu.SemaphoreType.DMA,
        pltpu.SemaphoreType.DMA,
        pltpu.SemaphoreType.DMA,
    ],
)
def _embed_sc(idxT_hbm, w128_hbm, out4_hbm, idx_s, pidx_v, par_v, pair_v,
              otile_v, g0, g1, o0, o1):
    wid = lax.axis_index("s") * NC + lax.axis_index("c")
    bcol0 = wid * (BPW * BB)
    lanes = lax.iota(jnp.int32, 16)

    gsems = (g0, g1)
    osems = (o0, o1)

    def stage_idx(h0):
        pltpu.sync_copy(
            idxT_hbm.at[pl.ds(pl.multiple_of(h0, 8), 8),
                        pl.ds(pl.multiple_of(bcol0, BB), BPW * BB)], idx_s)

    def prep_block(t, s):
        h8 = lax.rem(lax.div(t, BPW), jnp.int32(8))
        c0 = lax.rem(t, BPW) * BB
        for lg in range(8):
            iv = idx_s[h8, pl.ds(c0 + 16 * lg, 16)]
            pidx_v[s, pl.ds(16 * lg, 16)] = lax.shift_right_logical(iv, 1)
            par_v[s, lg] = lax.bitwise_and(iv, 1) * D

    def fire_gather(s):
        pltpu.async_copy(w128_hbm.at[pidx_v.at[s]], pair_v.at[s], gsems[s])

    def drain_gather(s):
        pltpu.make_async_copy(w128_hbm.at[pl.ds(0, BB)],
                              pair_v.at[s], gsems[s]).wait()

    def extract(s):
        for lg in range(8):
            rows = 16 * lg + lanes
            par = par_v[s, lg]
            for dg in range(8):
                for dm in range(8):
                    col = par + (dg * 8 + dm)
                    otile_v[s, dg, dm, pl.ds(16 * lg, 16)] = (
                        plsc.load_gather(pair_v.at[s], [rows, col]))

    def fire_out(t, s):
        h = lax.div(t, BPW)
        bb = lax.rem(t, BPW) + lax.div(bcol0, BB)
        for dg in range(8):
            pltpu.async_copy(otile_v.at[s, dg],
                             out4_hbm.at[h * 8 + dg, bb], osems[s])

    def drain_out(s):
        for dg in range(8):
            pltpu.make_async_copy(otile_v.at[s, dg],
                                  out4_hbm.at[0, 0], osems[s]).wait()

    stage_idx(0)
    prep_block(0, 0)
    fire_gather(0)

    def body(g, carry):
        for u in range(2):
            t = 2 * g + u
            s = u
            @pl.when(t + 1 < NBLK)
            def _():
                @pl.when(lax.rem(t + 1, 8 * BPW) == 0)
                def _():
                    stage_idx(lax.div(t + 1, BPW))
                prep_block(t + 1, 1 - s)
                fire_gather(1 - s)

            drain_gather(s)

            @pl.when(t >= 2)
            def _():
                drain_out(s)
            extract(s)
            fire_out(t, s)
        return carry

    lax.fori_loop(0, NBLK // 2, body, 0)
    drain_out(0)
    drain_out(1)


def kernel(input_, weight):
    idxT = jnp.pad(input_.T, ((0, HPAD - HIST), (0, 0)))
    w128 = weight.reshape(NUM_EMBEDDINGS // 2, 2 * D)
    out4 = _embed_sc(idxT, w128)
    out = (out4.reshape(HIST, 8, BATCH // BB, 8, BB)
           .transpose(2, 4, 0, 1, 3)
           .reshape(BATCH, HIST, D))
    return out


# speedup vs baseline: 1.5613x; 1.5613x over previous
import functools

import jax
import jax.numpy as jnp
from jax import lax
from jax.experimental import pallas as pl
from jax.experimental.pallas import tpu as pltpu
from jax.experimental.pallas import tpu_sc as plsc

NUM_EMBEDDINGS = 1000000
EMBEDDING_DIM = 64
BATCH = 16384
HIST = 50

NC, NS = 2, 16
NW = NC * NS
D = EMBEDDING_DIM
BLK = 512
NT = BATCH * HIST // NW // BLK
GPB = BLK // 128

_mesh = plsc.VectorSubcoreMesh(core_axis_name="c", subcore_axis_name="s",
                               num_cores=NC, num_subcores=NS)


@functools.partial(
    pl.kernel,
    out_type=jax.ShapeDtypeStruct((NW, NT, GPB, 128, D), jnp.float32),
    mesh=_mesh,
    compiler_params=pltpu.CompilerParams(use_tc_tiling_on_sc=False),
    scratch_types=[
        pltpu.VMEM((NT, BLK), jnp.int32),
        pltpu.VMEM((2, GPB, 128), jnp.int32),
        pltpu.VMEM((2, GPB, 128, D), jnp.float32),
        pltpu.SemaphoreType.DMA,
        pltpu.SemaphoreType.DMA,
        pltpu.SemaphoreType.DMA,
        pltpu.SemaphoreType.DMA,
    ],
)
def _embed_sc(idx_hbm, w_hbm, out_hbm, idx_s, bidx_v, rows_v, g0, g1, o0, o1):
    wid = lax.axis_index("s") * NC + lax.axis_index("c")
    gsems = (g0, g1)
    osems = (o0, o1)

    def prep_block(t, s):
        for j in range(GPB):
            for q in range(8):
                bidx_v[s, j, pl.ds(16 * q, 16)] = (
                    idx_s[t, pl.ds(j * 128 + 16 * q, 16)])

    def fire_gather(s):
        for j in range(GPB):
            pltpu.async_copy(w_hbm.at[bidx_v.at[s, j]], rows_v.at[s, j],
                             gsems[s])

    def drain_gather(s):
        for j in range(GPB):
            pltpu.make_async_copy(w_hbm.at[pl.ds(0, 128)], rows_v.at[s, j],
                                  gsems[s]).wait()

    def fire_out(t, s):
        pltpu.async_copy(rows_v.at[s], out_hbm.at[wid, t], osems[s])

    def drain_out(s):
        pltpu.make_async_copy(rows_v.at[s], out_hbm.at[0, 0], osems[s]).wait()

    pltpu.sync_copy(idx_hbm.at[wid], idx_s)
    prep_block(0, 0)
    fire_gather(0)

    def body(g, carry):
        for u in range(2):
            t = 2 * g + u
            s = u
            @pl.when(t + 1 < NT)
            def _():
                @pl.when(t >= 1)
                def _():
                    drain_out(1 - s)
                prep_block(t + 1, 1 - s)
                fire_gather(1 - s)
            drain_gather(s)
            fire_out(t, s)
        return carry

    lax.fori_loop(0, NT // 2, body, 0)
    drain_out(0)
    drain_out(1)


def kernel(input_, weight):
    idx3 = input_.reshape(NW, NT, BLK)
    out5 = _embed_sc(idx3, weight)
    return out5.reshape(BATCH, HIST, D)
